# baseline (speedup 1.0000x reference)
import functools

import jax
import jax.numpy as jnp
from jax import lax
from jax.experimental import pallas as pl
from jax.experimental.pallas import tpu as pltpu
from jax.experimental.pallas import tpu_sc as plsc

_N = 10000
_NP = 10240
_E = 320000
_D = 128
_C = 128
_ALPHA = 0.2
_NC = 2
_NS = 16
_NW = _NC * _NS
_CH = 128
_NCHUNK = _E // _CH
_NA = 10112
_RPT = _NA // _NS

_DCF = _NCHUNK // _NW
_DCX = _NCHUNK - _DCF * _NW

_vmesh = plsc.VectorSubcoreMesh(core_axis_name="c", subcore_axis_name="s")


@functools.partial(
    pl.kernel,
    out_type=jax.ShapeDtypeStruct((2, _NW, _NP), jnp.int32),
    mesh=_vmesh,
    scratch_types=[
        pltpu.VMEM((2, (_DCF + 1) * _CH), jnp.int32),
        pltpu.VMEM((_NP,), jnp.int32),
    ],
    compiler_params=pltpu.CompilerParams(needs_layout_passes=False),
)
def _degrees(edge_hbm, deg_out, idx_v, hist_v):
    wid = lax.axis_index("s") * _NC + lax.axis_index("c")
    col0 = _DCF * wid + jnp.minimum(wid, _DCX)
    ones = jnp.ones((16,), jnp.int32)
    zeros = jnp.zeros((16,), jnp.int32)

    pltpu.sync_copy(edge_hbm.at[:, pl.ds(col0 * _CH, _DCF * _CH)],
                    idx_v.at[:, pl.ds(0, _DCF * _CH)])

    @pl.when(wid < _DCX)
    def _extra():
        pltpu.sync_copy(edge_hbm.at[:, pl.ds((col0 + _DCF) * _CH, _CH)],
                        idx_v.at[:, pl.ds(_DCF * _CH, _CH)])

    ngrp4 = (_DCF * _CH // 64) + jnp.where(wid < _DCX, _CH // 64, 0)
    for which in range(2):

        def _zero(i, carry):
            for u in range(8):
                hist_v[pl.ds((i * 8 + u) * 16, 16)] = zeros
            return carry

        lax.fori_loop(0, _NP // 128, _zero, None)

        def _acc(i, carry):
            for u in range(4):
                v = idx_v[which, pl.ds((i * 4 + u) * 16, 16)]
                plsc.addupdate_scatter(hist_v, [v], ones)
            return carry

        lax.fori_loop(0, ngrp4, _acc, None)
        pltpu.sync_copy(hist_v, deg_out.at[which, wid])


@functools.partial(
    pl.kernel,
    out_type=jax.ShapeDtypeStruct((_NC, _NA, _D), jnp.float32),
    mesh=_vmesh,
    scratch_types=[
        pltpu.VMEM_SHARED((_NA, _D), jnp.float32),
        pltpu.VMEM((4, 2, _CH), jnp.int32),
        pltpu.VMEM((3, _CH, _D), jnp.float32),
        pltpu.SemaphoreType.DMA((4,)),
        pltpu.SemaphoreType.DMA((3,)),
        pltpu.SemaphoreType.DMA((3,)),
    ],
)
def _propagate(feat_hbm, edge_hbm, agg_out,
               agg_sh, eidx_v, rows_v, isems, gsems, ssems):
    cid = lax.axis_index("c")
    sid = lax.axis_index("s")
    wid = sid * _NC + cid

    zeros = jnp.zeros((16,), jnp.float32)

    def _zrow(r, carry):
        for c8 in range(_D // 16):
            rows_v[0, r, pl.ds(c8 * 16, 16)] = zeros
        return carry

    lax.fori_loop(0, _CH, _zrow, None)

    tile_base = sid * _RPT
    for j in range(_RPT // _CH):
        pltpu.sync_copy(rows_v.at[0], agg_sh.at[pl.ds(tile_base + j * _CH, _CH)])
    _rtail = _RPT - (_RPT // _CH) * _CH
    if _rtail:
        pltpu.sync_copy(
            rows_v.at[0, pl.ds(0, _rtail)],
            agg_sh.at[pl.ds(tile_base + (_RPT // _CH) * _CH, _rtail)])
    plsc.subcore_barrier()

    nfull = _NCHUNK // _NW
    extra = _NCHUNK - nfull * _NW
    nmine = nfull + (wid < extra).astype(jnp.int32)

    def _idx_dma(i, slot):
        eb = (wid + i * _NW) * _CH
        return pltpu.make_async_copy(edge_hbm.at[:, pl.ds(eb, _CH)],
                                     eidx_v.at[slot], isems.at[slot])

    def _scatter_dma(i, rslot):
        return pltpu.make_async_copy(
            rows_v.at[rslot], agg_sh.at[eidx_v.at[lax.rem(i, 4), 1]],
            ssems.at[rslot])

    pltpu.sync_copy(edge_hbm.at[:, pl.ds(wid * _CH, _CH)], eidx_v.at[0])
    pltpu.async_copy(feat_hbm.at[eidx_v.at[0, 0]], rows_v.at[0], gsems.at[0])

    @pl.when(1 < nmine)
    def _pro1():
        pltpu.sync_copy(edge_hbm.at[:, pl.ds((wid + _NW) * _CH, _CH)],
                        eidx_v.at[1])

    def _chunk(i, carry):
        slot = lax.rem(i, 3)
        nslot = lax.rem(i + 1, 3)
        islot2 = lax.rem(i + 2, 4)
        islot1 = lax.rem(i + 1, 4)

        @pl.when(i >= 2)
        def _drain_scatter():
            _scatter_dma(i + 2, nslot).wait()

        @pl.when(i + 2 < nmine)
        def _issue_idx():
            _idx_dma(i + 2, islot2).start()

        @pl.when((i + 1 < nmine) & (i + 1 >= 2))
        def _wait_idx():
            _idx_dma(i + 1, islot1).wait()

        @pl.when(i + 1 < nmine)
        def _issue_gather():
            pltpu.async_copy(feat_hbm.at[eidx_v.at[islot1, 0]],
                             rows_v.at[nslot], gsems.at[nslot])

        pltpu.make_async_copy(feat_hbm.at[eidx_v.at[lax.rem(i, 4), 0]],
                              rows_v.at[slot], gsems.at[slot]).wait()
        _scatter_dma(i, slot).start(add=True)
        return carry

    lax.fori_loop(0, nmine, _chunk, None)

    @pl.when(nmine >= 2)
    def _ep0():
        _scatter_dma(nmine + 2, lax.rem(nmine + 1, 3)).wait()

    _scatter_dma(nmine + 3, lax.rem(nmine + 2, 3)).wait()
    plsc.subcore_barrier()
    pltpu.sync_copy(agg_sh.at[pl.ds(tile_base, _RPT)],
                    agg_out.at[cid, pl.ds(tile_base, _RPT)])


_BM = 2048


def _matmul_body(x_ref, w_ref, b_ref, deg_ref, h0_ref, feat_ref):
    h0 = lax.dot_general(x_ref[...], w_ref[...], (((1,), (1,)), ((), ())),
                         preferred_element_type=jnp.float32) + b_ref[...]
    deg = jnp.sum(deg_ref[0], axis=0)
    norm = lax.rsqrt(jnp.maximum(deg, 1).astype(jnp.float32))
    h0_ref[...] = h0
    feat_ref[...] = h0 * norm[:, None]


def _combine_body(h0_ref, agg_ref, deg_ref, out_ref):
    agg = agg_ref[0, :, :] + agg_ref[1, :, :]
    deg = jnp.sum(deg_ref[0], axis=0)
    norm = lax.rsqrt(jnp.maximum(deg, 1).astype(jnp.float32))
    out_ref[...] = (1.0 - _ALPHA) * agg * norm[:, None] + _ALPHA * h0_ref[...]


_matmul = pl.pallas_call(
    _matmul_body,
    grid=(_NP // _BM,),
    in_specs=[
        pl.BlockSpec((_BM, _D), lambda i: (i, 0)),
        pl.BlockSpec((_C, _D), lambda i: (0, 0)),
        pl.BlockSpec((1, _C), lambda i: (0, 0)),
        pl.BlockSpec((1, _NW, _BM), lambda i: (0, 0, i)),
    ],
    out_specs=[
        pl.BlockSpec((_BM, _C), lambda i: (i, 0)),
        pl.BlockSpec((_BM, _C), lambda i: (i, 0)),
    ],
    out_shape=[
        jax.ShapeDtypeStruct((_N, _C), jnp.float32),
        jax.ShapeDtypeStruct((_N, _C), jnp.float32),
    ],
)

_combine = pl.pallas_call(
    _combine_body,
    grid=(_NP // _BM,),
    in_specs=[
        pl.BlockSpec((_BM, _C), lambda i: (i, 0)),
        pl.BlockSpec((_NC, _BM, _C), lambda i: (0, i, 0)),
        pl.BlockSpec((1, _NW, _BM), lambda i: (1, 0, i)),
    ],
    out_specs=pl.BlockSpec((_BM, _C), lambda i: (i, 0)),
    out_shape=jax.ShapeDtypeStruct((_N, _C), jnp.float32),
)


def kernel(in_feat, edge_index, W, b):
    deg = _degrees(edge_index)
    h0, feat = _matmul(in_feat, W, b.reshape(1, _C), deg)
    agg = _propagate(feat, edge_index)
    return _combine(h0, agg, deg)


# speedup vs baseline: 1.0067x; 1.0067x over previous
import functools

import jax
import jax.numpy as jnp
from jax import lax
from jax.experimental import pallas as pl
from jax.experimental.pallas import tpu as pltpu
from jax.experimental.pallas import tpu_sc as plsc

_N = 10000
_NP = 10240
_E = 320000
_D = 128
_C = 128
_ALPHA = 0.2
_NC = 2
_NS = 16
_NW = _NC * _NS
_CH = 128
_NCHUNK = _E // _CH
_NA = 10112
_RPT = _NA // _NS

_DCF = _NCHUNK // _NS
_DCX = _NCHUNK - _DCF * _NS

_vmesh = plsc.VectorSubcoreMesh(core_axis_name="c", subcore_axis_name="s")


@functools.partial(
    pl.kernel,
    out_type=jax.ShapeDtypeStruct((2, _NS, _NP), jnp.int32),
    mesh=_vmesh,
    scratch_types=[
        pltpu.VMEM((2, (_DCF + 1) * _CH), jnp.int32),
        pltpu.VMEM((_NP,), jnp.int32),
    ],
    compiler_params=pltpu.CompilerParams(needs_layout_passes=False),
)
def _degrees(edge_hbm, deg_out, idx_v, hist_v):
    cid = lax.axis_index("c")
    sid = lax.axis_index("s")
    col0 = _DCF * sid + jnp.minimum(sid, _DCX)
    ones = jnp.ones((16,), jnp.int32)
    zeros = jnp.zeros((16,), jnp.int32)

    pltpu.sync_copy(edge_hbm.at[:, pl.ds(col0 * _CH, _DCF * _CH)],
                    idx_v.at[:, pl.ds(0, _DCF * _CH)])

    @pl.when(sid < _DCX)
    def _extra():
        pltpu.sync_copy(edge_hbm.at[:, pl.ds((col0 + _DCF) * _CH, _CH)],
                        idx_v.at[:, pl.ds(_DCF * _CH, _CH)])

    ngrp4 = (_DCF * _CH // 64) + jnp.where(sid < _DCX, _CH // 64, 0)

    def _zero(i, carry):
        for u in range(8):
            hist_v[pl.ds((i * 8 + u) * 16, 16)] = zeros
        return carry

    lax.fori_loop(0, _NP // 128, _zero, None)

    def _acc(i, carry):
        for u in range(4):
            v = idx_v[cid, pl.ds((i * 4 + u) * 16, 16)]
            plsc.addupdate_scatter(hist_v, [v], ones)
        return carry

    lax.fori_loop(0, ngrp4, _acc, None)
    pltpu.sync_copy(hist_v, deg_out.at[cid, sid])


@functools.partial(
    pl.kernel,
    out_type=jax.ShapeDtypeStruct((_NC, _NA, _D), jnp.float32),
    mesh=_vmesh,
    scratch_types=[
        pltpu.VMEM_SHARED((_NA, _D), jnp.float32),
        pltpu.VMEM((4, 2, _CH), jnp.int32),
        pltpu.VMEM((3, _CH, _D), jnp.float32),
        pltpu.SemaphoreType.DMA((4,)),
        pltpu.SemaphoreType.DMA((3,)),
        pltpu.SemaphoreType.DMA((3,)),
    ],
)
def _propagate(feat_hbm, edge_hbm, agg_out,
               agg_sh, eidx_v, rows_v, isems, gsems, ssems):
    cid = lax.axis_index("c")
    sid = lax.axis_index("s")
    wid = sid * _NC + cid

    zeros = jnp.zeros((16,), jnp.float32)

    def _zrow(r, carry):
        for c8 in range(_D // 16):
            rows_v[0, r, pl.ds(c8 * 16, 16)] = zeros
        return carry

    lax.fori_loop(0, _CH, _zrow, None)

    tile_base = sid * _RPT
    for j in range(_RPT // _CH):
        pltpu.sync_copy(rows_v.at[0], agg_sh.at[pl.ds(tile_base + j * _CH, _CH)])
    _rtail = _RPT - (_RPT // _CH) * _CH
    if _rtail:
        pltpu.sync_copy(
            rows_v.at[0, pl.ds(0, _rtail)],
            agg_sh.at[pl.ds(tile_base + (_RPT // _CH) * _CH, _rtail)])
    plsc.subcore_barrier()

    nfull = _NCHUNK // _NW
    extra = _NCHUNK - nfull * _NW
    nmine = nfull + (wid < extra).astype(jnp.int32)

    def _idx_dma(i, slot):
        eb = (wid + i * _NW) * _CH
        return pltpu.make_async_copy(edge_hbm.at[:, pl.ds(eb, _CH)],
                                     eidx_v.at[slot], isems.at[slot])

    def _scatter_dma(i, rslot):
        return pltpu.make_async_copy(
            rows_v.at[rslot], agg_sh.at[eidx_v.at[lax.rem(i, 4), 1]],
            ssems.at[rslot])

    pltpu.sync_copy(edge_hbm.at[:, pl.ds(wid * _CH, _CH)], eidx_v.at[0])
    pltpu.async_copy(feat_hbm.at[eidx_v.at[0, 0]], rows_v.at[0], gsems.at[0])

    @pl.when(1 < nmine)
    def _pro1():
        pltpu.sync_copy(edge_hbm.at[:, pl.ds((wid + _NW) * _CH, _CH)],
                        eidx_v.at[1])

    def _chunk(i, carry):
        slot = lax.rem(i, 3)
        nslot = lax.rem(i + 1, 3)
        islot2 = lax.rem(i + 2, 4)
        islot1 = lax.rem(i + 1, 4)

        @pl.when(i >= 2)
        def _drain_scatter():
            _scatter_dma(i + 2, nslot).wait()

        @pl.when(i + 2 < nmine)
        def _issue_idx():
            _idx_dma(i + 2, islot2).start()

        @pl.when((i + 1 < nmine) & (i + 1 >= 2))
        def _wait_idx():
            _idx_dma(i + 1, islot1).wait()

        @pl.when(i + 1 < nmine)
        def _issue_gather():
            pltpu.async_copy(feat_hbm.at[eidx_v.at[islot1, 0]],
                             rows_v.at[nslot], gsems.at[nslot])

        pltpu.make_async_copy(feat_hbm.at[eidx_v.at[lax.rem(i, 4), 0]],
                              rows_v.at[slot], gsems.at[slot]).wait()
        _scatter_dma(i, slot).start(add=True)
        return carry

    lax.fori_loop(0, nmine, _chunk, None)

    @pl.when(nmine >= 2)
    def _ep0():
        _scatter_dma(nmine + 2, lax.rem(nmine + 1, 3)).wait()

    _scatter_dma(nmine + 3, lax.rem(nmine + 2, 3)).wait()
    plsc.subcore_barrier()
    pltpu.sync_copy(agg_sh.at[pl.ds(tile_base, _RPT)],
                    agg_out.at[cid, pl.ds(tile_base, _RPT)])


_BM = 2048


def _matmul_body(x_ref, w_ref, b_ref, deg_ref, h0_ref, feat_ref):
    h0 = lax.dot_general(x_ref[...], w_ref[...], (((1,), (1,)), ((), ())),
                         preferred_element_type=jnp.float32) + b_ref[...]
    deg = jnp.sum(deg_ref[0], axis=0)
    norm = lax.rsqrt(jnp.maximum(deg, 1).astype(jnp.float32))
    h0_ref[...] = h0
    feat_ref[...] = h0 * norm[:, None]


def _combine_body(h0_ref, agg_ref, deg_ref, out_ref):
    agg = agg_ref[0, :, :] + agg_ref[1, :, :]
    deg = jnp.sum(deg_ref[0], axis=0)
    norm = lax.rsqrt(jnp.maximum(deg, 1).astype(jnp.float32))
    out_ref[...] = (1.0 - _ALPHA) * agg * norm[:, None] + _ALPHA * h0_ref[...]


_matmul = pl.pallas_call(
    _matmul_body,
    grid=(_NP // _BM,),
    in_specs=[
        pl.BlockSpec((_BM, _D), lambda i: (i, 0)),
        pl.BlockSpec((_C, _D), lambda i: (0, 0)),
        pl.BlockSpec((1, _C), lambda i: (0, 0)),
        pl.BlockSpec((1, _NS, _BM), lambda i: (0, 0, i)),
    ],
    out_specs=[
        pl.BlockSpec((_BM, _C), lambda i: (i, 0)),
        pl.BlockSpec((_BM, _C), lambda i: (i, 0)),
    ],
    out_shape=[
        jax.ShapeDtypeStruct((_N, _C), jnp.float32),
        jax.ShapeDtypeStruct((_N, _C), jnp.float32),
    ],
)

_combine = pl.pallas_call(
    _combine_body,
    grid=(_NP // _BM,),
    in_specs=[
        pl.BlockSpec((_BM, _C), lambda i: (i, 0)),
        pl.BlockSpec((_NC, _BM, _C), lambda i: (0, i, 0)),
        pl.BlockSpec((1, _NS, _BM), lambda i: (1, 0, i)),
    ],
    out_specs=pl.BlockSpec((_BM, _C), lambda i: (i, 0)),
    out_shape=jax.ShapeDtypeStruct((_N, _C), jnp.float32),
)


def kernel(in_feat, edge_index, W, b):
    deg = _degrees(edge_index)
    h0, feat = _matmul(in_feat, W, b.reshape(1, _C), deg)
    agg = _propagate(feat, edge_index)
    return _combine(h0, agg, deg)


# speedup vs baseline: 1.0082x; 1.0014x over previous
import functools

import jax
import jax.numpy as jnp
from jax import lax
from jax.experimental import pallas as pl
from jax.experimental.pallas import tpu as pltpu
from jax.experimental.pallas import tpu_sc as plsc

_N = 10000
_NP = 10240
_E = 320000
_D = 128
_C = 128
_ALPHA = 0.2
_NC = 2
_NS = 16
_NW = _NC * _NS
_CH = 128
_NCHUNK = _E // _CH
_NA = 10112
_RPT = _NA // _NS

_DCF = _NCHUNK // _NS
_DCX = _NCHUNK - _DCF * _NS

_vmesh = plsc.VectorSubcoreMesh(core_axis_name="c", subcore_axis_name="s")


@functools.partial(
    pl.kernel,
    out_type=jax.ShapeDtypeStruct((2, _NS, _NP), jnp.int32),
    mesh=_vmesh,
    scratch_types=[
        pltpu.VMEM((2, (_DCF + 1) * _CH), jnp.int32),
        pltpu.VMEM((_NP,), jnp.int32),
    ],
    compiler_params=pltpu.CompilerParams(needs_layout_passes=False),
)
def _degrees(edge_hbm, deg_out, idx_v, hist_v):
    cid = lax.axis_index("c")
    sid = lax.axis_index("s")
    col0 = _DCF * sid + jnp.minimum(sid, _DCX)
    ones = jnp.ones((16,), jnp.int32)
    zeros = jnp.zeros((16,), jnp.int32)

    pltpu.sync_copy(edge_hbm.at[:, pl.ds(col0 * _CH, _DCF * _CH)],
                    idx_v.at[:, pl.ds(0, _DCF * _CH)])

    @pl.when(sid < _DCX)
    def _extra():
        pltpu.sync_copy(edge_hbm.at[:, pl.ds((col0 + _DCF) * _CH, _CH)],
                        idx_v.at[:, pl.ds(_DCF * _CH, _CH)])

    ngrp4 = (_DCF * _CH // 64) + jnp.where(sid < _DCX, _CH // 64, 0)

    def _zero(i, carry):
        for u in range(8):
            hist_v[pl.ds((i * 8 + u) * 16, 16)] = zeros
        return carry

    lax.fori_loop(0, _NP // 128, _zero, None)

    def _acc(i, carry):
        for u in range(4):
            v = idx_v[cid, pl.ds((i * 4 + u) * 16, 16)]
            plsc.addupdate_scatter(hist_v, [v], ones)
        return carry

    lax.fori_loop(0, ngrp4, _acc, None)
    pltpu.sync_copy(hist_v, deg_out.at[cid, sid])


@functools.partial(
    pl.kernel,
    out_type=jax.ShapeDtypeStruct((_NC, _NA, _D), jnp.float32),
    mesh=_vmesh,
    scratch_types=[
        pltpu.VMEM_SHARED((_NA, _D), jnp.float32),
        pltpu.VMEM((4, 2, _CH), jnp.int32),
        pltpu.VMEM((3, _CH, _D), jnp.float32),
        pltpu.SemaphoreType.DMA((4,)),
        pltpu.SemaphoreType.DMA((3,)),
        pltpu.SemaphoreType.DMA((3,)),
    ],
)
def _propagate(feat_hbm, edge_hbm, agg_out,
               agg_sh, eidx_v, rows_v, isems, gsems, ssems):
    cid = lax.axis_index("c")
    sid = lax.axis_index("s")
    wid = sid * _NC + cid

    zeros = jnp.zeros((16,), jnp.float32)

    nfull = _NCHUNK // _NW
    extra = _NCHUNK - nfull * _NW
    nmine = nfull + (wid < extra).astype(jnp.int32)

    def _idx_dma(i, slot):
        eb = (wid + i * _NW) * _CH
        return pltpu.make_async_copy(edge_hbm.at[:, pl.ds(eb, _CH)],
                                     eidx_v.at[slot], isems.at[slot])

    def _scatter_dma(i, rslot):
        return pltpu.make_async_copy(
            rows_v.at[rslot], agg_sh.at[eidx_v.at[lax.rem(i, 4), 1]],
            ssems.at[rslot])

    pltpu.sync_copy(edge_hbm.at[:, pl.ds(wid * _CH, _CH)], eidx_v.at[0])
    pltpu.async_copy(feat_hbm.at[eidx_v.at[0, 0]], rows_v.at[0], gsems.at[0])

    @pl.when(1 < nmine)
    def _pro1():
        pltpu.sync_copy(edge_hbm.at[:, pl.ds((wid + _NW) * _CH, _CH)],
                        eidx_v.at[1])

    def _zrow(r, carry):
        for c8 in range(_D // 16):
            rows_v[2, r, pl.ds(c8 * 16, 16)] = zeros
        return carry

    lax.fori_loop(0, _CH, _zrow, None)

    tile_base = sid * _RPT
    for j in range(_RPT // _CH):
        pltpu.sync_copy(rows_v.at[2], agg_sh.at[pl.ds(tile_base + j * _CH, _CH)])
    _rtail = _RPT - (_RPT // _CH) * _CH
    if _rtail:
        pltpu.sync_copy(
            rows_v.at[2, pl.ds(0, _rtail)],
            agg_sh.at[pl.ds(tile_base + (_RPT // _CH) * _CH, _rtail)])
    plsc.subcore_barrier()

    def _chunk(i, carry):
        slot = lax.rem(i, 3)
        nslot = lax.rem(i + 1, 3)
        islot2 = lax.rem(i + 2, 4)
        islot1 = lax.rem(i + 1, 4)

        @pl.when(i >= 2)
        def _drain_scatter():
            _scatter_dma(i + 2, nslot).wait()

        @pl.when(i + 2 < nmine)
        def _issue_idx():
            _idx_dma(i + 2, islot2).start()

        @pl.when((i + 1 < nmine) & (i + 1 >= 2))
        def _wait_idx():
            _idx_dma(i + 1, islot1).wait()

        @pl.when(i + 1 < nmine)
        def _issue_gather():
            pltpu.async_copy(feat_hbm.at[eidx_v.at[islot1, 0]],
                             rows_v.at[nslot], gsems.at[nslot])

        pltpu.make_async_copy(feat_hbm.at[eidx_v.at[lax.rem(i, 4), 0]],
                              rows_v.at[slot], gsems.at[slot]).wait()
        _scatter_dma(i, slot).start(add=True)
        return carry

    lax.fori_loop(0, nmine, _chunk, None)

    @pl.when(nmine >= 2)
    def _ep0():
        _scatter_dma(nmine + 2, lax.rem(nmine + 1, 3)).wait()

    _scatter_dma(nmine + 3, lax.rem(nmine + 2, 3)).wait()
    plsc.subcore_barrier()
    pltpu.sync_copy(agg_sh.at[pl.ds(tile_base, _RPT)],
                    agg_out.at[cid, pl.ds(tile_base, _RPT)])


_BM = 2048


def _matmul_body(x_ref, w_ref, b_ref, deg_ref, h0_ref, feat_ref):
    h0 = lax.dot_general(x_ref[...], w_ref[...], (((1,), (1,)), ((), ())),
                         preferred_element_type=jnp.float32) + b_ref[...]
    deg = jnp.sum(deg_ref[0], axis=0)
    norm = lax.rsqrt(jnp.maximum(deg, 1).astype(jnp.float32))
    h0_ref[...] = h0
    feat_ref[...] = h0 * norm[:, None]


def _combine_body(h0_ref, agg_ref, deg_ref, out_ref):
    agg = agg_ref[0, :, :] + agg_ref[1, :, :]
    deg = jnp.sum(deg_ref[0], axis=0)
    norm = lax.rsqrt(jnp.maximum(deg, 1).astype(jnp.float32))
    out_ref[...] = (1.0 - _ALPHA) * agg * norm[:, None] + _ALPHA * h0_ref[...]


_matmul = pl.pallas_call(
    _matmul_body,
    grid=(_NP // _BM,),
    in_specs=[
        pl.BlockSpec((_BM, _D), lambda i: (i, 0)),
        pl.BlockSpec((_C, _D), lambda i: (0, 0)),
        pl.BlockSpec((1, _C), lambda i: (0, 0)),
        pl.BlockSpec((1, _NS, _BM), lambda i: (0, 0, i)),
    ],
    out_specs=[
        pl.BlockSpec((_BM, _C), lambda i: (i, 0)),
        pl.BlockSpec((_BM, _C), lambda i: (i, 0)),
    ],
    out_shape=[
        jax.ShapeDtypeStruct((_N, _C), jnp.float32),
        jax.ShapeDtypeStruct((_N, _C), jnp.float32),
    ],
)

_combine = pl.pallas_call(
    _combine_body,
    grid=(_NP // _BM,),
    in_specs=[
        pl.BlockSpec((_BM, _C), lambda i: (i, 0)),
        pl.BlockSpec((_NC, _BM, _C), lambda i: (0, i, 0)),
        pl.BlockSpec((1, _NS, _BM), lambda i: (1, 0, i)),
    ],
    out_specs=pl.BlockSpec((_BM, _C), lambda i: (i, 0)),
    out_shape=jax.ShapeDtypeStruct((_N, _C), jnp.float32),
)


def kernel(in_feat, edge_index, W, b):
    deg = _degrees(edge_index)
    h0, feat = _matmul(in_feat, W, b.reshape(1, _C), deg)
    agg = _propagate(feat, edge_index)
    return _combine(h0, agg, deg)
